# Initial kernel scaffold; baseline (speedup 1.0000x reference)
import jax
import jax.numpy as jnp
from jax.experimental import pallas as pl


def kernel(feats, segment_ids, W, b):
    raise NotImplementedError("write your pallas kernel here")


# speedup vs baseline: 4.0337x; 4.0337x over previous
import jax
import jax.numpy as jnp
from jax import lax
from jax.experimental import pallas as pl
from jax.experimental.pallas import tpu as pltpu

N = 100000
F = 128
G = 512
B = 2000
NB = N // B


def _body(seg_ref, x_ref, wrow_ref, b_ref, out_ref, aw_ref):
    i = pl.program_id(0)
    x = x_ref[...]
    wrow = wrow_ref[...]
    aw = jnp.sum(x * wrow, axis=1, keepdims=True) + b_ref[0, 0]
    aw_ref[...] = aw
    w = jax.nn.sigmoid(aw)
    weighted = (x * w).astype(jnp.bfloat16)
    ids = seg_ref[0, 0, :]
    onehot = (ids[:, None] == lax.broadcasted_iota(jnp.int32, (B, G), 1))
    onehot = onehot.astype(jnp.bfloat16)
    partial = lax.dot_general(
        onehot, weighted,
        dimension_numbers=(((0,), (0,)), ((), ())),
        preferred_element_type=jnp.float32,
    )

    @pl.when(i == 0)
    def _():
        out_ref[...] = jnp.zeros_like(out_ref)

    out_ref[...] += partial


def kernel(feats, segment_ids, W, b):
    seg = segment_ids.astype(jnp.int32).reshape(NB, 1, B)
    wrow = W.reshape(1, F)
    b2 = b.reshape(1, 1)
    out, aw = pl.pallas_call(
        _body,
        grid=(NB,),
        in_specs=[
            pl.BlockSpec((1, 1, B), lambda i: (i, 0, 0)),
            pl.BlockSpec((B, F), lambda i: (i, 0)),
            pl.BlockSpec((1, F), lambda i: (0, 0)),
            pl.BlockSpec((1, 1), lambda i: (0, 0)),
        ],
        out_specs=[
            pl.BlockSpec((G, F), lambda i: (0, 0)),
            pl.BlockSpec((B, 1), lambda i: (i, 0)),
        ],
        out_shape=[
            jax.ShapeDtypeStruct((G, F), jnp.float32),
            jax.ShapeDtypeStruct((N, 1), jnp.float32),
        ],
    )(seg, feats, wrow, b2)
    return (out, aw)
